# baseline (speedup 1.0000x reference)
import functools

import jax
import jax.numpy as jnp
from jax import lax
from jax.experimental import pallas as pl
from jax.experimental.pallas import tpu as pltpu
from jax.experimental.pallas import tpu_sc as plsc

_BATCH = 64
_NPOS = 576
_DIM = 768
_NC = 2
_NS = 16
_NW = _NC * _NS
_PW = _NPOS // _NW
_BLK = _PW * _DIM
_L = 16
_NCH = _BLK // _L
_IMG = _NPOS * _DIM
_GB = 2
_NGRP = _BATCH // _GB
_NBUF = 2

_mesh = plsc.VectorSubcoreMesh(
    core_axis_name="c", subcore_axis_name="s", num_cores=_NC)


@functools.partial(
    pl.kernel,
    out_type=jax.ShapeDtypeStruct((_BATCH, _IMG), jnp.float32),
    mesh=_mesh,
    scratch_types=[
        pltpu.VMEM((_BLK,), jnp.float32),
        [pltpu.VMEM((_GB, _BLK), jnp.float32) for _ in range(_NBUF)],
        [pltpu.VMEM((_GB, _BLK), jnp.float32) for _ in range(_NBUF)],
        [pltpu.SemaphoreType.DMA for _ in range(_NBUF)],
        [pltpu.SemaphoreType.DMA for _ in range(_NBUF)],
    ],
)
def _pos_add(patches_hbm, pos_hbm, out_hbm, pos_v, ibufs, obufs, isems, osems):
    wid = lax.axis_index("s") * _NC + lax.axis_index("c")
    base = wid * _BLK
    pltpu.sync_copy(pos_hbm.at[pl.ds(base, _BLK)], pos_v)

    def start_in(g, j):
        pltpu.async_copy(
            patches_hbm.at[pl.ds(g * _GB, _GB), pl.ds(base, _BLK)],
            ibufs[j], isems[j])

    for j in range(_NBUF):
        start_in(j, j)

    @pl.loop(0, _NGRP, step=_NBUF)
    def _group(g0):
        for j in range(_NBUF):
            g = g0 + j
            rows = pl.ds(g * _GB, _GB)
            cols = pl.ds(base, _BLK)
            pltpu.make_async_copy(
                patches_hbm.at[rows, cols], ibufs[j], isems[j]).wait()

            @pl.when(g >= _NBUF)
            def _():
                pltpu.make_async_copy(
                    obufs[j], out_hbm.at[rows, cols], osems[j]).wait()

            for b2 in range(_GB):
                @plsc.parallel_loop(0, _NCH, unroll=16)
                def _chunk(i):
                    sl = pl.ds(i * _L, _L)
                    obufs[j].at[b2][sl] = ibufs[j].at[b2][sl] + pos_v[sl]

            pltpu.async_copy(obufs[j], out_hbm.at[rows, cols], osems[j])

            @pl.when(g + _NBUF < _NGRP)
            def _():
                start_in(g + _NBUF, j)

    for j in range(_NBUF):
        pltpu.make_async_copy(
            obufs[j], out_hbm.at[pl.ds(0, _GB), pl.ds(0, _BLK)],
            osems[j]).wait()


def kernel(patches, pos_table):
    out = _pos_add(patches.reshape(_BATCH, _IMG), pos_table.reshape(-1))
    return out.reshape(_BATCH, _NPOS, _DIM)


# speedup vs baseline: 1.0049x; 1.0049x over previous
import functools

import jax
import jax.numpy as jnp
from jax import lax
from jax.experimental import pallas as pl
from jax.experimental.pallas import tpu as pltpu
from jax.experimental.pallas import tpu_sc as plsc

_BATCH = 64
_NPOS = 576
_DIM = 768
_NC = 2
_NS = 16
_NW = _NC * _NS
_PW = _NPOS // _NW
_BLK = _PW * _DIM
_L = 16
_NCH = _BLK // _L
_IMG = _NPOS * _DIM
_GB = 2
_NGRP = _BATCH // _GB
_NBUF = 2

_mesh = plsc.VectorSubcoreMesh(
    core_axis_name="c", subcore_axis_name="s", num_cores=_NC)


@functools.partial(
    pl.kernel,
    out_type=jax.ShapeDtypeStruct((_BATCH, _IMG), jnp.float32),
    mesh=_mesh,
    scratch_types=[
        pltpu.VMEM((_BLK,), jnp.float32),
        [pltpu.VMEM((_GB, _BLK), jnp.float32) for _ in range(_NBUF)],
        [pltpu.VMEM((_GB, _BLK), jnp.float32) for _ in range(_NBUF)],
        [pltpu.SemaphoreType.DMA for _ in range(_NBUF)],
        [pltpu.SemaphoreType.DMA for _ in range(_NBUF)],
    ],
)
def _pos_add(patches_hbm, pos_hbm, out_hbm, pos_v, ibufs, obufs, isems, osems):
    wid = lax.axis_index("s") * _NC + lax.axis_index("c")
    base = wid * _BLK
    pos_copy = pltpu.async_copy(
        pos_hbm.at[pl.ds(base, _BLK)], pos_v, osems[0])

    def start_in(g, j):
        pltpu.async_copy(
            patches_hbm.at[pl.ds(g * _GB, _GB), pl.ds(base, _BLK)],
            ibufs[j], isems[j])

    for j in range(_NBUF):
        start_in(j, j)
    pos_copy.wait()

    @pl.loop(0, _NGRP, step=_NBUF)
    def _group(g0):
        for j in range(_NBUF):
            g = g0 + j
            rows = pl.ds(g * _GB, _GB)
            cols = pl.ds(base, _BLK)
            pltpu.make_async_copy(
                patches_hbm.at[rows, cols], ibufs[j], isems[j]).wait()

            @pl.when(g >= _NBUF)
            def _():
                pltpu.make_async_copy(
                    obufs[j], out_hbm.at[rows, cols], osems[j]).wait()

            for b2 in range(_GB):
                @plsc.parallel_loop(0, _NCH, unroll=16)
                def _chunk(i):
                    sl = pl.ds(i * _L, _L)
                    obufs[j].at[b2][sl] = ibufs[j].at[b2][sl] + pos_v[sl]

            pltpu.async_copy(obufs[j], out_hbm.at[rows, cols], osems[j])

            @pl.when(g + _NBUF < _NGRP)
            def _():
                start_in(g + _NBUF, j)

    for j in range(_NBUF):
        pltpu.make_async_copy(
            obufs[j], out_hbm.at[pl.ds(0, _GB), pl.ds(0, _BLK)],
            osems[j]).wait()


def kernel(patches, pos_table):
    out = _pos_add(patches.reshape(_BATCH, _IMG), pos_table.reshape(-1))
    return out.reshape(_BATCH, _NPOS, _DIM)
